# baseline (speedup 1.0000x reference)
import functools

import jax
import jax.numpy as jnp
from jax import lax
from jax.experimental import pallas as pl
from jax.experimental.pallas import tpu as pltpu
from jax.experimental.pallas import tpu_sc as plsc

_K = 8
_M = 100000
_D = 128
_N = 16384
_NW = 32
_NBLKS = 6272
_MPAD = _NBLKS * 16
_BPW = _NBLKS // _NW
_CHB = 28
_NCH = _BPW // _CHB
_CHW = _CHB * 16 * _D


def _sc_scan(dict_blocked, resid_bcast):
    mesh = plsc.VectorSubcoreMesh(core_axis_name="c", subcore_axis_name="s")

    @functools.partial(
        pl.kernel,
        mesh=mesh,
        out_type=(
            jax.ShapeDtypeStruct((_NW, 16), jnp.float32),
            jax.ShapeDtypeStruct((_NW, 16), jnp.int32),
        ),
        scratch_types=[
            pltpu.VMEM((_D * 16,), jnp.float32),
            pltpu.VMEM((_CHW,), jnp.float32),
            pltpu.VMEM((_CHW,), jnp.float32),
            pltpu.VMEM((16,), jnp.float32),
            pltpu.VMEM((16,), jnp.int32),
            pltpu.SemaphoreType.DMA,
            pltpu.SemaphoreType.DMA,
        ],
        compiler_params=pltpu.CompilerParams(needs_layout_passes=False),
    )
    def scan_kernel(d_hbm, rb_hbm, val_out, idx_out, rb_v, buf0, buf1,
                    vout_v, iout_v, sem0, sem1):
        wid = lax.axis_index("s") * 2 + lax.axis_index("c")
        pltpu.sync_copy(rb_hbm, rb_v)
        iota16 = lax.broadcasted_iota(jnp.int32, (16,), 0)
        blk0 = wid * _BPW

        def dma_start(ch, buf, sem):
            off = (blk0 + ch * _CHB) * (_D * 16)
            pltpu.make_async_copy(
                d_hbm.at[pl.ds(off, _CHW)], buf, sem).start()

        def dma_wait(buf, sem):
            pltpu.make_async_copy(
                d_hbm.at[pl.ds(0, _CHW)], buf, sem).wait()

        def chunk_compute(buf, ch, bv, bi):
            nquad = 4

            def quad_body(p, carry2):
                bv2, bi2 = carry2
                boffs = [p * (nquad * 16 * _D) + q * (16 * _D)
                         for q in range(nquad)]
                accs = [[None] * 4 for _ in range(nquad)]
                for c in range(_D):
                    rv = rb_v[pl.ds(c * 16, 16)]
                    k = c & 3
                    for q in range(nquad):
                        dq = buf[pl.ds(boffs[q] + c * 16, 16)]
                        pq = dq * rv
                        accs[q][k] = (pq if accs[q][k] is None
                                      else accs[q][k] + pq)
                gblk = blk0 + ch * _CHB + nquad * p
                for q in range(nquad):
                    aq = jnp.abs((accs[q][0] + accs[q][1])
                                 + (accs[q][2] + accs[q][3]))
                    rows_q = (gblk + q) * 16 + iota16
                    pred = aq > bv2
                    bv2 = jnp.where(pred, aq, bv2)
                    bi2 = jnp.where(pred, rows_q, bi2)
                return bv2, bi2

            return lax.fori_loop(0, _CHB // nquad, quad_body, (bv, bi))

        bv = jnp.broadcast_to(jnp.float32(-1.0), (16,))
        bi = jnp.broadcast_to(jnp.int32(0), (16,))
        dma_start(0, buf0, sem0)

        def pair_of_chunks(i, carry):
            bv2, bi2 = carry
            dma_wait(buf0, sem0)

            @pl.when(2 * i + 1 < _NCH)
            def _():
                dma_start(2 * i + 1, buf1, sem1)

            bv2, bi2 = chunk_compute(buf0, 2 * i, bv2, bi2)

            @pl.when(2 * i + 2 < _NCH)
            def _():
                dma_start(2 * i + 2, buf0, sem0)

            @pl.when(2 * i + 1 < _NCH)
            def _2():
                dma_wait(buf1, sem1)

            bv3, bi3 = chunk_compute(buf1, 2 * i + 1, bv2, bi2)
            use_b = 2 * i + 1 < _NCH
            bv2 = jnp.where(use_b, bv3, bv2)
            bi2 = jnp.where(use_b, bi3, bi2)
            return bv2, bi2

        bv, bi = lax.fori_loop(0, (_NCH + 1) // 2, pair_of_chunks, (bv, bi))

        vout_v[...] = bv
        iout_v[...] = bi
        pltpu.sync_copy(vout_v, val_out.at[wid])
        pltpu.sync_copy(iout_v, idx_out.at[wid])

    return scan_kernel(dict_blocked, resid_bcast)


def _rowcontract(a, b):
    return lax.dot_general(a, b, (((1,), (1,)), ((), ())),
                           preferred_element_type=jnp.float32,
                           precision=lax.Precision.HIGHEST)


def _make_tc_update(j):
    def body(vals_ref, idxs_ref, xpc_ref, g_ref, b_ref, at_ref, dict_ref,
             resid_o, recon_o, rn_o, absw_o, g_o, b_o, at_o, idx_o,
             atom_scr, sem):
        i32 = jnp.int32
        r88 = lax.broadcasted_iota(i32, (_K, _K), 0)
        c88 = lax.broadcasted_iota(i32, (_K, _K), 1)
        r81 = lax.broadcasted_iota(i32, (_K, 1), 0)

        vals = vals_ref[...]
        idxs = idxs_ref[...]
        mx = jnp.max(vals)
        gidx = jnp.min(jnp.where(vals >= mx, idxs, i32(2147483647)))
        idx_o[...] = jnp.reshape(gidx, (1, 1))

        start = pl.multiple_of((gidx // 8) * 8, 8)
        cp = pltpu.make_async_copy(dict_ref.at[pl.ds(start, 8)], atom_scr, sem)
        cp.start()
        cp.wait()
        rsel = (lax.broadcasted_iota(i32, (8, 1), 0)
                == (gidx - start)).astype(jnp.float32)
        atom = jnp.sum(atom_scr[...] * rsel, axis=0, keepdims=True)

        at_new = jnp.where(r81 == j, atom, at_ref[...])
        dots_col = _rowcontract(at_new, atom)
        dots_row = _rowcontract(atom, at_new)
        g = g_ref[...]
        g = jnp.where(r88 == j, dots_row, g)
        g = jnp.where(c88 == j, dots_col, g)
        xpc = xpc_ref[...]
        bj = _rowcontract(atom, xpc)
        b = jnp.where(r81 == j, bj, b_ref[...])

        m = g
        y = b
        for k in range(_K):
            mrow = m[k:k + 1, :]
            piv = m[k:k + 1, k:k + 1]
            yk = y[k:k + 1, :]
            fcol = m[:, k:k + 1] / piv
            fm = jnp.where(r81 > k, fcol, 0.0)
            m = m - fm * mrow
            y = y - fm * yk
        w = jnp.zeros((_K, 1), jnp.float32)
        for k in range(_K - 1, -1, -1):
            mrow = m[k:k + 1, :]
            piv = m[k:k + 1, k:k + 1]
            yk = y[k:k + 1, :]
            wm = jnp.where(r81 > k, w, 0.0)
            s = yk - lax.dot_general(
                mrow, wm, (((1,), (0,)), ((), ())),
                preferred_element_type=jnp.float32,
                precision=lax.Precision.HIGHEST)
            w = jnp.where(r81 == k, s / piv, w)

        recon = lax.dot_general(
            w, at_new, (((0,), (0,)), ((), ())),
            preferred_element_type=jnp.float32,
            precision=lax.Precision.HIGHEST)
        resid_o[...] = xpc - recon
        recon_o[...] = recon
        nrmsq = jnp.sum(recon * recon, axis=1, keepdims=True)
        rn_o[...] = recon / jnp.sqrt(nrmsq)
        absw_o[...] = jnp.abs(w)
        g_o[...] = g
        b_o[...] = b
        at_o[...] = at_new

    f32 = jnp.float32
    return pl.pallas_call(
        body,
        out_shape=(
            jax.ShapeDtypeStruct((1, _D), f32),
            jax.ShapeDtypeStruct((1, _D), f32),
            jax.ShapeDtypeStruct((1, _D), f32),
            jax.ShapeDtypeStruct((_K, 1), f32),
            jax.ShapeDtypeStruct((_K, _K), f32),
            jax.ShapeDtypeStruct((_K, 1), f32),
            jax.ShapeDtypeStruct((_K, _D), f32),
            jax.ShapeDtypeStruct((1, 1), jnp.int32),
        ),
        in_specs=[
            pl.BlockSpec(memory_space=pltpu.MemorySpace.VMEM),
            pl.BlockSpec(memory_space=pltpu.MemorySpace.VMEM),
            pl.BlockSpec(memory_space=pltpu.MemorySpace.VMEM),
            pl.BlockSpec(memory_space=pltpu.MemorySpace.VMEM),
            pl.BlockSpec(memory_space=pltpu.MemorySpace.VMEM),
            pl.BlockSpec(memory_space=pltpu.MemorySpace.VMEM),
            pl.BlockSpec(memory_space=pl.ANY),
        ],
        scratch_shapes=[
            pltpu.VMEM((8, _D), f32),
            pltpu.SemaphoreType.DMA,
        ],
    )


_ROWS_BLK = 512
_NBLK = _N // _ROWS_BLK


def _tc_stats(X, rn_mat, chosen, descriptors):
    f32 = jnp.float32

    def body(x_ref, rn_ref, chosen_ref, desc_ref,
             evr_o, l2_o, cos_o, res_o,
             st_s, st2_s, scos_s, colsum_s, sx2_s, desc_scr, sem):
        pid = pl.program_id(0)

        @pl.when(pid == 0)
        def _init():
            st_s[...] = jnp.zeros_like(st_s)
            st2_s[...] = jnp.zeros_like(st2_s)
            scos_s[...] = jnp.zeros_like(scos_s)
            colsum_s[...] = jnp.zeros_like(colsum_s)
            sx2_s[0] = 0.0

        xb = x_ref[...]
        rn = rn_ref[...]
        rnsq = jnp.sum(rn * rn, axis=1, keepdims=True)
        nrm_row = jnp.sqrt(jnp.reshape(rnsq, (1, _K)))
        tb = jnp.dot(xb, rn.T, preferred_element_type=f32,
                     precision=lax.Precision.HIGHEST)
        rowsq = jnp.sum(xb * xb, axis=1, keepdims=True)
        rown = jnp.sqrt(rowsq)
        t2 = tb * tb
        den = (jnp.maximum(rown, 1e-8)
               * jnp.maximum(jnp.abs(tb) * nrm_row, 1e-8))
        st_s[...] = st_s[...] + jnp.sum(tb, axis=0, keepdims=True)
        st2_s[...] = st2_s[...] + jnp.sum(t2, axis=0, keepdims=True)
        scos_s[...] = scos_s[...] + jnp.sum(t2 / den, axis=0, keepdims=True)
        colsum_s[...] = colsum_s[...] + jnp.sum(xb, axis=0, keepdims=True)
        sx2_s[0] = sx2_s[0] + jnp.sum(rowsq)

        @pl.when(pid == _NBLK - 1)
        def _fin():
            n = f32(_N)
            st = st_s[...]
            st2 = st2_s[...]
            scos = scos_s[...]
            colsum = colsum_s[...]
            sx2 = sx2_s[0]
            rnsq_row = jnp.reshape(rnsq, (1, _K))
            var_t = (st2 - st * st / n) / (n - 1.0)
            std_orig = (sx2 - jnp.sum(colsum * colsum) / n) / (n - 1.0)
            evr_o[...] = var_t * rnsq_row / std_orig
            l2_o[...] = (sx2 - 2.0 * st2 + st2 * rnsq_row) / (n * f32(_D))
            cos_o[...] = scos / n
            c18 = lax.broadcasted_iota(jnp.int32, (1, _K), 1)
            c1d = lax.broadcasted_iota(jnp.int32, (1, _D), 1)
            res = jnp.zeros((1, _K), jnp.int32)
            for k in range(_K):
                ck = chosen_ref[k]
                start_k = pl.multiple_of((ck // _D) * _D, _D)
                cpk = pltpu.make_async_copy(
                    desc_ref.at[pl.ds(start_k, _D)], desc_scr, sem)
                cpk.start()
                cpk.wait()
                win = jnp.reshape(desc_scr[...], (1, _D))
                val = jnp.sum(jnp.where(c1d == (ck - start_k), win, 0))
                res = jnp.where(c18 == k, val, res)
            res_o[...] = res

    return pl.pallas_call(
        body,
        grid=(_NBLK,),
        out_shape=(
            jax.ShapeDtypeStruct((1, _K), f32),
            jax.ShapeDtypeStruct((1, _K), f32),
            jax.ShapeDtypeStruct((1, _K), f32),
            jax.ShapeDtypeStruct((1, _K), jnp.int32),
        ),
        in_specs=[
            pl.BlockSpec((_ROWS_BLK, _D), lambda i: (i, 0)),
            pl.BlockSpec((_K, _D), lambda i: (0, 0)),
            pl.BlockSpec(memory_space=pltpu.MemorySpace.SMEM),
            pl.BlockSpec(memory_space=pl.ANY),
        ],
        out_specs=(
            pl.BlockSpec((1, _K), lambda i: (0, 0)),
            pl.BlockSpec((1, _K), lambda i: (0, 0)),
            pl.BlockSpec((1, _K), lambda i: (0, 0)),
            pl.BlockSpec((1, _K), lambda i: (0, 0)),
        ),
        scratch_shapes=[
            pltpu.VMEM((1, _K), f32),
            pltpu.VMEM((1, _K), f32),
            pltpu.VMEM((1, _K), f32),
            pltpu.VMEM((1, _D), f32),
            pltpu.SMEM((1,), f32),
            pltpu.VMEM((_D,), jnp.int32),
            pltpu.SemaphoreType.DMA,
        ],
    )(X, rn_mat, chosen, descriptors)


def kernel(X, dictionary, descriptors, device):
    x_mean = jnp.mean(X, axis=0)
    xc = X - x_mean
    _, _, vt = jnp.linalg.svd(xc, full_matrices=False)
    x_pc = vt[0]

    dict_blocked = (
        jnp.concatenate(
            [dictionary,
             jnp.zeros((_MPAD - _M, _D), dictionary.dtype)], axis=0)
        .reshape(_NBLKS, 16, _D).swapaxes(1, 2).reshape(-1))
    g = jnp.eye(_K, dtype=jnp.float32)
    b = jnp.zeros((_K, 1), jnp.float32)
    at = jnp.zeros((_K, _D), jnp.float32)
    resid2d = x_pc.reshape(1, _D)
    chosen_parts = []
    rn_parts = []
    recon2d = jnp.zeros((1, _D), jnp.float32)
    absw = jnp.zeros((_K, 1), jnp.float32)
    xpc2d = x_pc.reshape(1, _D)
    for j in range(_K):
        rb = jnp.broadcast_to(resid2d.reshape(_D, 1), (_D, 16)).reshape(-1)
        vals, idxs = _sc_scan(dict_blocked, rb)
        (resid2d, recon2d, rn_j, absw, g, b, at, idx_j) = _make_tc_update(j)(
            vals, idxs, xpc2d, g, b, at, dictionary)
        chosen_parts.append(idx_j.reshape(1))
        rn_parts.append(rn_j)
    chosen_arr = jnp.concatenate(chosen_parts)
    rn_mat = jnp.concatenate(rn_parts, axis=0)
    desc_pad = jnp.concatenate([descriptors, jnp.zeros((_D,), descriptors.dtype)])
    evr, l2, cosine, results = _tc_stats(X, rn_mat, chosen_arr, desc_pad)
    return (recon2d.reshape(_D), results.reshape(_K), chosen_arr,
            absw.reshape(_K), evr.reshape(_K), l2.reshape(_K),
            cosine.reshape(_K))


# speedup vs baseline: 1.0049x; 1.0049x over previous
import functools

import jax
import jax.numpy as jnp
from jax import lax
from jax.experimental import pallas as pl
from jax.experimental.pallas import tpu as pltpu
from jax.experimental.pallas import tpu_sc as plsc

_K = 8
_M = 100000
_D = 128
_N = 16384
_NW = 32
_NBLKS = 6272
_MPAD = _NBLKS * 16
_BPW = _NBLKS // _NW
_CHB = 14
_NCH = _BPW // _CHB
_CHW = _CHB * 16 * _D


def _sc_scan(dict_blocked, resid_bcast):
    mesh = plsc.VectorSubcoreMesh(core_axis_name="c", subcore_axis_name="s")

    @functools.partial(
        pl.kernel,
        mesh=mesh,
        out_type=(
            jax.ShapeDtypeStruct((_NW, 16), jnp.float32),
            jax.ShapeDtypeStruct((_NW, 16), jnp.int32),
        ),
        scratch_types=[
            pltpu.VMEM((_D * 16,), jnp.float32),
            pltpu.VMEM((_CHW,), jnp.float32),
            pltpu.VMEM((_CHW,), jnp.float32),
            pltpu.VMEM((16,), jnp.float32),
            pltpu.VMEM((16,), jnp.int32),
            pltpu.SemaphoreType.DMA,
            pltpu.SemaphoreType.DMA,
        ],
        compiler_params=pltpu.CompilerParams(needs_layout_passes=False),
    )
    def scan_kernel(d_hbm, rb_hbm, val_out, idx_out, rb_v, buf0, buf1,
                    vout_v, iout_v, sem0, sem1):
        wid = lax.axis_index("s") * 2 + lax.axis_index("c")
        pltpu.sync_copy(rb_hbm, rb_v)
        iota16 = lax.broadcasted_iota(jnp.int32, (16,), 0)
        blk0 = wid * _BPW

        def dma_start(ch, buf, sem):
            off = (blk0 + ch * _CHB) * (_D * 16)
            pltpu.make_async_copy(
                d_hbm.at[pl.ds(off, _CHW)], buf, sem).start()

        def dma_wait(buf, sem):
            pltpu.make_async_copy(
                d_hbm.at[pl.ds(0, _CHW)], buf, sem).wait()

        def chunk_compute(buf, ch, bv, bi):
            def pair_body(p, carry2):
                bv2, bi2 = carry2
                boff_a = p * (2 * 16 * _D)
                boff_b = boff_a + 16 * _D
                acc_a = [None] * 4
                acc_b = [None] * 4
                for c in range(_D):
                    rv = rb_v[pl.ds(c * 16, 16)]
                    da = buf[pl.ds(boff_a + c * 16, 16)]
                    db = buf[pl.ds(boff_b + c * 16, 16)]
                    k = c & 3
                    pa = da * rv
                    pb = db * rv
                    acc_a[k] = pa if acc_a[k] is None else acc_a[k] + pa
                    acc_b[k] = pb if acc_b[k] is None else acc_b[k] + pb
                a_a = jnp.abs((acc_a[0] + acc_a[1]) + (acc_a[2] + acc_a[3]))
                a_b = jnp.abs((acc_b[0] + acc_b[1]) + (acc_b[2] + acc_b[3]))
                gblk = blk0 + ch * _CHB + 2 * p
                rows_a = gblk * 16 + iota16
                rows_b = rows_a + 16
                pred = a_a > bv2
                bv2 = jnp.where(pred, a_a, bv2)
                bi2 = jnp.where(pred, rows_a, bi2)
                pred = a_b > bv2
                bv2 = jnp.where(pred, a_b, bv2)
                bi2 = jnp.where(pred, rows_b, bi2)
                return bv2, bi2

            return lax.fori_loop(0, _CHB // 2, pair_body, (bv, bi))

        bv = jnp.broadcast_to(jnp.float32(-1.0), (16,))
        bi = jnp.broadcast_to(jnp.int32(0), (16,))
        dma_start(0, buf0, sem0)

        def pair_of_chunks(i, carry):
            bv2, bi2 = carry
            dma_wait(buf0, sem0)

            @pl.when(2 * i + 1 < _NCH)
            def _():
                dma_start(2 * i + 1, buf1, sem1)

            bv2, bi2 = chunk_compute(buf0, 2 * i, bv2, bi2)

            @pl.when(2 * i + 2 < _NCH)
            def _():
                dma_start(2 * i + 2, buf0, sem0)

            @pl.when(2 * i + 1 < _NCH)
            def _2():
                dma_wait(buf1, sem1)

            bv3, bi3 = chunk_compute(buf1, 2 * i + 1, bv2, bi2)
            use_b = 2 * i + 1 < _NCH
            bv2 = jnp.where(use_b, bv3, bv2)
            bi2 = jnp.where(use_b, bi3, bi2)
            return bv2, bi2

        bv, bi = lax.fori_loop(0, (_NCH + 1) // 2, pair_of_chunks, (bv, bi))

        vout_v[...] = bv
        iout_v[...] = bi
        pltpu.sync_copy(vout_v, val_out.at[wid])
        pltpu.sync_copy(iout_v, idx_out.at[wid])

    return scan_kernel(dict_blocked, resid_bcast)


def _rowcontract(a, b):
    return lax.dot_general(a, b, (((1,), (1,)), ((), ())),
                           preferred_element_type=jnp.float32,
                           precision=lax.Precision.HIGHEST)


def _make_tc_update(j):
    def body(vals_ref, idxs_ref, xpc_ref, g_ref, b_ref, at_ref, dict_ref,
             resid_o, recon_o, rn_o, absw_o, g_o, b_o, at_o, idx_o,
             atom_scr, sem):
        i32 = jnp.int32
        r88 = lax.broadcasted_iota(i32, (_K, _K), 0)
        c88 = lax.broadcasted_iota(i32, (_K, _K), 1)
        r81 = lax.broadcasted_iota(i32, (_K, 1), 0)

        vals = vals_ref[...]
        idxs = idxs_ref[...]
        mx = jnp.max(vals)
        gidx = jnp.min(jnp.where(vals >= mx, idxs, i32(2147483647)))
        idx_o[...] = jnp.reshape(gidx, (1, 1))

        start = pl.multiple_of((gidx // 8) * 8, 8)
        cp = pltpu.make_async_copy(dict_ref.at[pl.ds(start, 8)], atom_scr, sem)
        cp.start()
        cp.wait()
        rsel = (lax.broadcasted_iota(i32, (8, 1), 0)
                == (gidx - start)).astype(jnp.float32)
        atom = jnp.sum(atom_scr[...] * rsel, axis=0, keepdims=True)

        at_new = jnp.where(r81 == j, atom, at_ref[...])
        dots_col = _rowcontract(at_new, atom)
        dots_row = _rowcontract(atom, at_new)
        g = g_ref[...]
        g = jnp.where(r88 == j, dots_row, g)
        g = jnp.where(c88 == j, dots_col, g)
        xpc = xpc_ref[...]
        bj = _rowcontract(atom, xpc)
        b = jnp.where(r81 == j, bj, b_ref[...])

        m = g
        y = b
        for k in range(_K):
            mrow = m[k:k + 1, :]
            piv = m[k:k + 1, k:k + 1]
            yk = y[k:k + 1, :]
            fcol = m[:, k:k + 1] / piv
            fm = jnp.where(r81 > k, fcol, 0.0)
            m = m - fm * mrow
            y = y - fm * yk
        w = jnp.zeros((_K, 1), jnp.float32)
        for k in range(_K - 1, -1, -1):
            mrow = m[k:k + 1, :]
            piv = m[k:k + 1, k:k + 1]
            yk = y[k:k + 1, :]
            wm = jnp.where(r81 > k, w, 0.0)
            s = yk - lax.dot_general(
                mrow, wm, (((1,), (0,)), ((), ())),
                preferred_element_type=jnp.float32,
                precision=lax.Precision.HIGHEST)
            w = jnp.where(r81 == k, s / piv, w)

        recon = lax.dot_general(
            w, at_new, (((0,), (0,)), ((), ())),
            preferred_element_type=jnp.float32,
            precision=lax.Precision.HIGHEST)
        resid_o[...] = xpc - recon
        recon_o[...] = recon
        nrmsq = jnp.sum(recon * recon, axis=1, keepdims=True)
        rn_o[...] = recon / jnp.sqrt(nrmsq)
        absw_o[...] = jnp.abs(w)
        g_o[...] = g
        b_o[...] = b
        at_o[...] = at_new

    f32 = jnp.float32
    return pl.pallas_call(
        body,
        out_shape=(
            jax.ShapeDtypeStruct((1, _D), f32),
            jax.ShapeDtypeStruct((1, _D), f32),
            jax.ShapeDtypeStruct((1, _D), f32),
            jax.ShapeDtypeStruct((_K, 1), f32),
            jax.ShapeDtypeStruct((_K, _K), f32),
            jax.ShapeDtypeStruct((_K, 1), f32),
            jax.ShapeDtypeStruct((_K, _D), f32),
            jax.ShapeDtypeStruct((1, 1), jnp.int32),
        ),
        in_specs=[
            pl.BlockSpec(memory_space=pltpu.MemorySpace.VMEM),
            pl.BlockSpec(memory_space=pltpu.MemorySpace.VMEM),
            pl.BlockSpec(memory_space=pltpu.MemorySpace.VMEM),
            pl.BlockSpec(memory_space=pltpu.MemorySpace.VMEM),
            pl.BlockSpec(memory_space=pltpu.MemorySpace.VMEM),
            pl.BlockSpec(memory_space=pltpu.MemorySpace.VMEM),
            pl.BlockSpec(memory_space=pl.ANY),
        ],
        scratch_shapes=[
            pltpu.VMEM((8, _D), f32),
            pltpu.SemaphoreType.DMA,
        ],
    )


_ROWS_BLK = 512
_NBLK = _N // _ROWS_BLK


def _tc_stats(X, rn_mat, chosen, descriptors):
    f32 = jnp.float32

    def body(x_ref, rn_ref, chosen_ref, desc_ref,
             evr_o, l2_o, cos_o, res_o,
             st_s, st2_s, scos_s, colsum_s, sx2_s, desc_scr, sem):
        pid = pl.program_id(0)

        @pl.when(pid == 0)
        def _init():
            st_s[...] = jnp.zeros_like(st_s)
            st2_s[...] = jnp.zeros_like(st2_s)
            scos_s[...] = jnp.zeros_like(scos_s)
            colsum_s[...] = jnp.zeros_like(colsum_s)
            sx2_s[0] = 0.0

        xb = x_ref[...]
        rn = rn_ref[...]
        rnsq = jnp.sum(rn * rn, axis=1, keepdims=True)
        nrm_row = jnp.sqrt(jnp.reshape(rnsq, (1, _K)))
        tb = jnp.dot(xb, rn.T, preferred_element_type=f32,
                     precision=lax.Precision.HIGHEST)
        rowsq = jnp.sum(xb * xb, axis=1, keepdims=True)
        rown = jnp.sqrt(rowsq)
        t2 = tb * tb
        den = (jnp.maximum(rown, 1e-8)
               * jnp.maximum(jnp.abs(tb) * nrm_row, 1e-8))
        st_s[...] = st_s[...] + jnp.sum(tb, axis=0, keepdims=True)
        st2_s[...] = st2_s[...] + jnp.sum(t2, axis=0, keepdims=True)
        scos_s[...] = scos_s[...] + jnp.sum(t2 / den, axis=0, keepdims=True)
        colsum_s[...] = colsum_s[...] + jnp.sum(xb, axis=0, keepdims=True)
        sx2_s[0] = sx2_s[0] + jnp.sum(rowsq)

        @pl.when(pid == _NBLK - 1)
        def _fin():
            n = f32(_N)
            st = st_s[...]
            st2 = st2_s[...]
            scos = scos_s[...]
            colsum = colsum_s[...]
            sx2 = sx2_s[0]
            rnsq_row = jnp.reshape(rnsq, (1, _K))
            var_t = (st2 - st * st / n) / (n - 1.0)
            std_orig = (sx2 - jnp.sum(colsum * colsum) / n) / (n - 1.0)
            evr_o[...] = var_t * rnsq_row / std_orig
            l2_o[...] = (sx2 - 2.0 * st2 + st2 * rnsq_row) / (n * f32(_D))
            cos_o[...] = scos / n
            c18 = lax.broadcasted_iota(jnp.int32, (1, _K), 1)
            c1d = lax.broadcasted_iota(jnp.int32, (1, _D), 1)
            res = jnp.zeros((1, _K), jnp.int32)
            for k in range(_K):
                ck = chosen_ref[k]
                start_k = pl.multiple_of((ck // _D) * _D, _D)
                cpk = pltpu.make_async_copy(
                    desc_ref.at[pl.ds(start_k, _D)], desc_scr, sem)
                cpk.start()
                cpk.wait()
                win = jnp.reshape(desc_scr[...], (1, _D))
                val = jnp.sum(jnp.where(c1d == (ck - start_k), win, 0))
                res = jnp.where(c18 == k, val, res)
            res_o[...] = res

    return pl.pallas_call(
        body,
        grid=(_NBLK,),
        out_shape=(
            jax.ShapeDtypeStruct((1, _K), f32),
            jax.ShapeDtypeStruct((1, _K), f32),
            jax.ShapeDtypeStruct((1, _K), f32),
            jax.ShapeDtypeStruct((1, _K), jnp.int32),
        ),
        in_specs=[
            pl.BlockSpec((_ROWS_BLK, _D), lambda i: (i, 0)),
            pl.BlockSpec((_K, _D), lambda i: (0, 0)),
            pl.BlockSpec(memory_space=pltpu.MemorySpace.SMEM),
            pl.BlockSpec(memory_space=pl.ANY),
        ],
        out_specs=(
            pl.BlockSpec((1, _K), lambda i: (0, 0)),
            pl.BlockSpec((1, _K), lambda i: (0, 0)),
            pl.BlockSpec((1, _K), lambda i: (0, 0)),
            pl.BlockSpec((1, _K), lambda i: (0, 0)),
        ),
        scratch_shapes=[
            pltpu.VMEM((1, _K), f32),
            pltpu.VMEM((1, _K), f32),
            pltpu.VMEM((1, _K), f32),
            pltpu.VMEM((1, _D), f32),
            pltpu.SMEM((1,), f32),
            pltpu.VMEM((_D,), jnp.int32),
            pltpu.SemaphoreType.DMA,
        ],
    )(X, rn_mat, chosen, descriptors)


def kernel(X, dictionary, descriptors, device):
    x_mean = jnp.mean(X, axis=0)
    xc = X - x_mean
    _, _, vt = jnp.linalg.svd(xc, full_matrices=False)
    x_pc = vt[0]

    dict_blocked = (
        jnp.concatenate(
            [dictionary,
             jnp.zeros((_MPAD - _M, _D), dictionary.dtype)], axis=0)
        .reshape(_NBLKS, 16, _D).swapaxes(1, 2).reshape(-1))
    g = jnp.eye(_K, dtype=jnp.float32)
    b = jnp.zeros((_K, 1), jnp.float32)
    at = jnp.zeros((_K, _D), jnp.float32)
    resid2d = x_pc.reshape(1, _D)
    chosen_parts = []
    rn_parts = []
    recon2d = jnp.zeros((1, _D), jnp.float32)
    absw = jnp.zeros((_K, 1), jnp.float32)
    xpc2d = x_pc.reshape(1, _D)
    for j in range(_K):
        rb = jnp.broadcast_to(resid2d.reshape(_D, 1), (_D, 16)).reshape(-1)
        vals, idxs = _sc_scan(dict_blocked, rb)
        (resid2d, recon2d, rn_j, absw, g, b, at, idx_j) = _make_tc_update(j)(
            vals, idxs, xpc2d, g, b, at, dictionary)
        chosen_parts.append(idx_j.reshape(1))
        rn_parts.append(rn_j)
    chosen_arr = jnp.concatenate(chosen_parts)
    rn_mat = jnp.concatenate(rn_parts, axis=0)
    desc_pad = jnp.concatenate([descriptors, jnp.zeros((_D,), descriptors.dtype)])
    evr, l2, cosine, results = _tc_stats(X, rn_mat, chosen_arr, desc_pad)
    return (recon2d.reshape(_D), results.reshape(_K), chosen_arr,
            absw.reshape(_K), evr.reshape(_K), l2.reshape(_K),
            cosine.reshape(_K))
